# baseline (speedup 1.0000x reference)
import jax
import jax.numpy as jnp
from jax import lax
from jax.experimental import pallas as pl
from jax.experimental.pallas import tpu as pltpu
from jax.experimental.pallas import tpu_sc as plsc

_NC = 2
_NS = 16
_NW = _NC * _NS
_LANES = 16
_G = 2
_NBUF = 4


def _make_body(batch, seq, embed, bc):
  n_groups = bc // _LANES
  n_stages = seq // _G

  def body(x_hbm, table_hbm, out_hbm, xbuf, idx_bufs, row_bufs, gsems, wsems):
    wid = lax.axis_index("s") * _NC + lax.axis_index("c")
    b0 = wid * bc
    pltpu.sync_copy(x_hbm.at[pl.ds(b0 * seq, bc * seq)], xbuf)

    lane = lax.iota(jnp.int32, _LANES)

    def build_idx(t, k):
      s0 = t * _G
      for i in range(_G):
        for j in range(n_groups):
          pos = (j * _LANES) * seq + lane * seq + (s0 + i)
          idx_bufs[k][pl.ds(i * bc + j * _LANES, _LANES)] = (
              plsc.load_gather(xbuf, [pos]))

    def start_gather(t, k):
      build_idx(t, k)
      pltpu.async_copy(table_hbm.at[idx_bufs[k]], row_bufs[k], gsems[k])

    def wait_gather(k):
      pltpu.make_async_copy(
          table_hbm.at[idx_bufs[k]], row_bufs[k], gsems[k]).wait()

    def write_descs(t, k):
      for i in range(_G):
        yield (row_bufs[k].at[pl.ds(i * bc, bc)],
               out_hbm.at[pl.ds((t * _G + i) * batch + b0, bc)], wsems[k])

    def fire_writes(t, k):
      for src, dst, sem in write_descs(t, k):
        pltpu.async_copy(src, dst, sem)

    def wait_writes(t, k):
      for src, dst, sem in write_descs(t, k):
        pltpu.make_async_copy(src, dst, sem).wait()

    for k in range(_NBUF):
      start_gather(k, k)

    def step(g, carry):
      for k in range(_NBUF):
        t = g * _NBUF + k
        kprev = (k - 1) % _NBUF
        wait_gather(k)
        fire_writes(t, k)

        @pl.when((t >= 1) & (t + _NBUF - 1 < n_stages))
        def _():
          wait_writes(t - 1, kprev)
          start_gather(t - 1 + _NBUF, kprev)
      return carry

    lax.fori_loop(0, n_stages // _NBUF, step, None)

    for k in range(_NBUF):
      wait_writes(n_stages - _NBUF + k, k)

  return body


@jax.jit
def kernel(x, table):
  batch, seq = x.shape
  _, embed = table.shape
  bc = batch // _NW
  x_flat = x.reshape(-1)

  mesh = plsc.VectorSubcoreMesh(core_axis_name="c", subcore_axis_name="s")
  out = pl.kernel(
      _make_body(batch, seq, embed, bc),
      out_type=jax.ShapeDtypeStruct((seq * batch, embed), jnp.float32),
      mesh=mesh,
      compiler_params=pltpu.CompilerParams(
          needs_layout_passes=False, use_tc_tiling_on_sc=False),
      scratch_types=[
          pltpu.VMEM((bc * seq,), jnp.int32),
          [pltpu.VMEM((_G * bc,), jnp.int32) for _ in range(_NBUF)],
          [pltpu.VMEM((_G * bc, embed), jnp.float32) for _ in range(_NBUF)],
          [pltpu.SemaphoreType.DMA for _ in range(_NBUF)],
          [pltpu.SemaphoreType.DMA for _ in range(_NBUF)],
      ],
  )(x_flat, table)
  return out.reshape(seq, batch, embed)
